# baseline (speedup 1.0000x reference)
import functools

import jax
import jax.numpy as jnp
from jax import lax
from jax.experimental import pallas as pl
from jax.experimental.pallas import tpu as pltpu
from jax.experimental.pallas import tpu_sc as plsc

B, L, H = 1024, 200, 64
N = B * L
EPS = 1e-12

_NC, _NS = 2, 16
NW = _NC * _NS
ROWS_PER_W = N // NW
GCH = 128
NG = ROWS_PER_W // GCH
SUP = 10
NSUP = NG // SUP
SROWS = SUP * GCH


def _sc_gather_body(ids_hbm, table_hbm, out_hbm, idx_v, rows_v, sem):
    wid = lax.axis_index("s") * _NC + lax.axis_index("c")
    pltpu.sync_copy(ids_hbm.at[wid], idx_v)

    def sup_body(s, carry):
        base = wid * ROWS_PER_W + s * SROWS
        cps = []
        for k in range(SUP):
            cp = pltpu.async_copy(
                table_hbm.at[idx_v.at[s * SUP + k]],
                rows_v.at[pl.ds(k * GCH, GCH)],
                sem,
            )
            cps.append(cp)
        for cp in cps:
            cp.wait()
        pltpu.sync_copy(rows_v, out_hbm.at[pl.ds(base, SROWS)])
        return carry

    lax.fori_loop(0, NSUP, sup_body, 0)


@functools.cache
def _sc_gather():
    return pl.kernel(
        _sc_gather_body,
        out_type=jax.ShapeDtypeStruct((N, H), jnp.float32),
        mesh=plsc.VectorSubcoreMesh(core_axis_name="c", subcore_axis_name="s"),
        compiler_params=pltpu.CompilerParams(use_tc_tiling_on_sc=False),
        scratch_types=[
            pltpu.VMEM((NG, GCH), jnp.int32),
            pltpu.VMEM((SROWS, H), jnp.float32),
            pltpu.SemaphoreType.DMA,
        ],
    )

R = 2048
NB = N // R


def _combine_body(g_ref, tt_ref, age_ref, ap_ref, seg_ref, wa_ref, ba_ref,
                  wp_ref, bp_ref, gam_ref, bet_ref, o_ref):
    g = g_ref[...]
    tt = tt_ref[...].reshape(R, 1)
    age = age_ref[...].reshape(R, 1)
    ap = ap_ref[...].reshape(R, 1)
    seg = jnp.where(tt == 0, seg_ref[0:1, :], seg_ref[1:2, :])
    hmask = lax.broadcasted_iota(jnp.int32, (1, H), 1) == 0
    va = age * wa_ref[...] + ba_ref[...]
    t2a = jnp.where(hmask, va, jnp.sin(va))
    vp = ap * wp_ref[...] + bp_ref[...]
    t2p = jnp.where(hmask, vp, jnp.sin(vp))
    emb = g + seg + t2a + t2p
    mu = jnp.mean(emb, axis=1, keepdims=True)
    d = emb - mu
    var = jnp.mean(d * d, axis=1, keepdims=True)
    o_ref[...] = d * lax.rsqrt(var + EPS) * gam_ref[...] + bet_ref[...]


_combine_specs = [
    pl.BlockSpec((R, H), lambda i: (i, 0)),
    pl.BlockSpec((1, 1, R), lambda i: (i, 0, 0)),
    pl.BlockSpec((1, 1, R), lambda i: (i, 0, 0)),
    pl.BlockSpec((1, 1, R), lambda i: (i, 0, 0)),
    pl.BlockSpec((2, H), lambda i: (0, 0)),
    pl.BlockSpec((1, H), lambda i: (0, 0)),
    pl.BlockSpec((1, H), lambda i: (0, 0)),
    pl.BlockSpec((1, H), lambda i: (0, 0)),
    pl.BlockSpec((1, H), lambda i: (0, 0)),
    pl.BlockSpec((1, H), lambda i: (0, 0)),
    pl.BlockSpec((1, H), lambda i: (0, 0)),
]

_combine = pl.pallas_call(
    _combine_body,
    grid=(NB,),
    in_specs=_combine_specs,
    out_specs=pl.BlockSpec((R, H), lambda i: (i, 0)),
    out_shape=jax.ShapeDtypeStruct((N, H), jnp.float32),
)


def kernel(input_ids, token_type_ids, age, abspos, concept_table,
           segment_table, age_w0, age_b0, age_w, age_b, abs_w0, abs_b0,
           abs_w, abs_b, ln_gamma, ln_beta):
    ids = input_ids.astype(jnp.int32).reshape(NW, NG, GCH)
    gathered = _sc_gather()(ids, concept_table)
    tt3 = token_type_ids.astype(jnp.int32).reshape(NB, 1, R)
    age3 = age.reshape(NB, 1, R)
    ap3 = abspos.reshape(NB, 1, R)
    wa = jnp.concatenate([age_w0, age_w]).reshape(1, H)
    ba = jnp.concatenate([age_b0, age_b]).reshape(1, H)
    wp = jnp.concatenate([abs_w0, abs_w]).reshape(1, H)
    bp = jnp.concatenate([abs_b0, abs_b]).reshape(1, H)
    gam = ln_gamma.reshape(1, H)
    bet = ln_beta.reshape(1, H)
    out = _combine(gathered, tt3, age3, ap3, segment_table,
                   wa, ba, wp, bp, gam, bet)
    return out.reshape(B, L, H)
